# Initial kernel scaffold; baseline (speedup 1.0000x reference)
import functools

import jax
import jax.numpy as jnp
from jax.experimental import pallas as pl
from jax.experimental.pallas import tpu as pltpu

_NUM_EXPERTS = 8
_TOP_K = 2
_NEG = jnp.float32(-1e30)


def _silu(v):
    return v * jax.nn.sigmoid(v)


def _moe_dense_body(x_ref, rw_ref, w1_ref, w2_ref, w3_ref, out_ref, gate_sc):
    e = pl.program_id(0)

    @pl.when(e == 0)
    def _compute_gates():
        x = x_ref[...]
        logits = jax.lax.dot_general(
            x, rw_ref[...], (((1,), (0,)), ((), ())),
            preferred_element_type=jnp.float32)
        col = jax.lax.broadcasted_iota(jnp.int32, logits.shape, 1)
        m1 = jnp.max(logits, axis=-1, keepdims=True)
        i1 = jnp.min(jnp.where(logits == m1, col, _NUM_EXPERTS),
                     axis=-1, keepdims=True)
        l2 = jnp.where(col == i1, _NEG, logits)
        m2 = jnp.max(l2, axis=-1, keepdims=True)
        i2 = jnp.min(jnp.where(l2 == m2, col, _NUM_EXPERTS),
                     axis=-1, keepdims=True)
        g1 = jax.nn.sigmoid(m1 - m2)
        g2 = 1.0 - g1
        gate_sc[...] = jnp.where(col == i1, g1, 0.0) + jnp.where(col == i2, g2, 0.0)

    x = x_ref[...]
    w1 = w1_ref[0]
    w3 = w3_ref[0]
    w2 = w2_ref[0]
    h = _silu(jax.lax.dot_general(x, w1, (((1,), (1,)), ((), ())),
                                  preferred_element_type=jnp.float32)) * \
        jax.lax.dot_general(x, w3, (((1,), (1,)), ((), ())),
                            preferred_element_type=jnp.float32)
    y = jax.lax.dot_general(h, w2, (((1,), (1,)), ((), ())),
                            preferred_element_type=jnp.float32)
    gate = gate_sc[:, pl.ds(e, 1)]
    prev = jnp.where(e == 0, jnp.zeros_like(y), out_ref[...])
    out_ref[...] = prev + gate * y


def kernel(x, router_w, w1, w2, w3):
    T, D = x.shape
    E, FF, _ = w1.shape
    grid = (E,)
    return pl.pallas_call(
        _moe_dense_body,
        grid=grid,
        in_specs=[
            pl.BlockSpec((T, D), lambda e: (0, 0)),
            pl.BlockSpec((D, E), lambda e: (0, 0)),
            pl.BlockSpec((1, FF, D), lambda e: (e, 0, 0)),
            pl.BlockSpec((1, D, FF), lambda e: (e, 0, 0)),
            pl.BlockSpec((1, FF, D), lambda e: (e, 0, 0)),
        ],
        out_specs=pl.BlockSpec((T, D), lambda e: (0, 0)),
        out_shape=jax.ShapeDtypeStruct((T, D), jnp.float32),
        scratch_shapes=[pltpu.VMEM((T, E), jnp.float32)],
    )(x, router_w, w1, w2, w3)


# speedup vs baseline: 2.0485x; 2.0485x over previous
import functools

import jax
import jax.numpy as jnp
from jax.experimental import pallas as pl
from jax.experimental.pallas import tpu as pltpu

_NUM_EXPERTS = 8
_TOP_K = 2
_NEG = -1e30


def _silu(v):
    return v * jax.nn.sigmoid(v)


def _moe_dense_body(x_ref, rw_ref, w1_ref, w2_ref, w3_ref, out_ref, gate_sc):
    e = pl.program_id(0)

    @pl.when(e == 0)
    def _compute_gates():
        x = x_ref[...]
        logits = jax.lax.dot_general(
            x, rw_ref[...], (((1,), (0,)), ((), ())),
            preferred_element_type=jnp.float32)
        col = jax.lax.broadcasted_iota(jnp.int32, logits.shape, 1)
        m1 = jnp.max(logits, axis=-1, keepdims=True)
        i1 = jnp.min(jnp.where(logits == m1, col, _NUM_EXPERTS),
                     axis=-1, keepdims=True)
        l2 = jnp.where(col == i1, _NEG, logits)
        m2 = jnp.max(l2, axis=-1, keepdims=True)
        i2 = jnp.min(jnp.where(l2 == m2, col, _NUM_EXPERTS),
                     axis=-1, keepdims=True)
        g1 = jax.nn.sigmoid(m1 - m2)
        g2 = 1.0 - g1
        gate_sc[...] = jnp.where(col == i1, g1, 0.0) + jnp.where(col == i2, g2, 0.0)

    x = x_ref[...]
    w1 = w1_ref[0]
    w3 = w3_ref[0]
    w2 = w2_ref[0]
    h = _silu(jax.lax.dot_general(x, w1, (((1,), (1,)), ((), ())),
                                  preferred_element_type=jnp.float32)) * \
        jax.lax.dot_general(x, w3, (((1,), (1,)), ((), ())),
                            preferred_element_type=jnp.float32)
    y = jax.lax.dot_general(h, w2, (((1,), (1,)), ((), ())),
                            preferred_element_type=jnp.float32)
    gate_all = gate_sc[...]
    colg = jax.lax.broadcasted_iota(jnp.int32, gate_all.shape, 1)
    gate = jnp.sum(jnp.where(colg == e, gate_all, 0.0), axis=-1, keepdims=True)
    prev = jnp.where(e == 0, jnp.zeros_like(y), out_ref[...])
    out_ref[...] = prev + gate * y


def kernel(x, router_w, w1, w2, w3):
    T, D = x.shape
    E, FF, _ = w1.shape
    grid = (E,)
    return pl.pallas_call(
        _moe_dense_body,
        grid=grid,
        in_specs=[
            pl.BlockSpec((T, D), lambda e: (0, 0)),
            pl.BlockSpec((D, E), lambda e: (0, 0)),
            pl.BlockSpec((1, FF, D), lambda e: (e, 0, 0)),
            pl.BlockSpec((1, D, FF), lambda e: (e, 0, 0)),
            pl.BlockSpec((1, FF, D), lambda e: (e, 0, 0)),
        ],
        out_specs=pl.BlockSpec((T, D), lambda e: (0, 0)),
        out_shape=jax.ShapeDtypeStruct((T, D), jnp.float32),
        scratch_shapes=[pltpu.VMEM((T, E), jnp.float32)],
    )(x, router_w, w1, w2, w3)
